# baseline (speedup 1.0000x reference)
import functools

import jax
import jax.numpy as jnp
from jax import lax
from jax.experimental import pallas as pl
from jax.experimental.pallas import tpu as pltpu
from jax.experimental.pallas import tpu_sc as plsc

B, H, N = 32, 8, 100000
L = 16
NC, NS = 2, 16
TB = 128
CTW = 20
WC = CTW * TB
RING = 3
NCH = 39
NROUND = NCH // RING
XTC = N - NCH * WC
GX = (NCH * WC) // L


@functools.partial(
    pl.kernel,
    mesh=plsc.VectorSubcoreMesh(core_axis_name="c", subcore_axis_name="s"),
    out_type=jax.ShapeDtypeStruct((B * L,), jnp.float32),
    scratch_types=[
        pltpu.VMEM((RING, H, WC), jnp.float32),
        pltpu.VMEM((H, XTC), jnp.float32),
        pltpu.VMEM((H, L), jnp.float32),
        pltpu.VMEM((H, L), jnp.int32),
        pltpu.VMEM((L,), jnp.float32),
        pltpu.SemaphoreType.DMA,
        pltpu.SemaphoreType.DMA,
        pltpu.SemaphoreType.DMA,
        pltpu.SemaphoreType.DMA,
    ],
)
def _argmax_sc(x_hbm, out_hbm, buf, tbuf, mbuf, abuf, res,
               sem0, sem1, sem2, semt):
    c = lax.axis_index("c")
    s = lax.axis_index("s")
    b = s * NC + c
    sems = (sem0, sem1, sem2)
    iota = lax.iota(jnp.int32, L)

    def chunk_copy(ci, slot):
        return pltpu.make_async_copy(
            x_hbm.at[b, :, pl.ds(ci * WC, WC)], buf.at[slot], sems[slot])

    for k in range(RING):
        chunk_copy(k, k).start()
    pltpu.make_async_copy(
        x_hbm.at[b, :, pl.ds(NCH * WC, XTC)], tbuf, semt).start()

    def scan_groups(bufref, gbase, ngroups, ms, aas):
        def body(g, carry):
            mm = list(carry[:H])
            aa = list(carry[H:])
            col = g * L
            gv = jnp.broadcast_to(gbase + g, (L,))
            for r in range(H):
                v = bufref[r, pl.ds(col, L)]
                gt = v > mm[r]
                mm[r] = jnp.where(gt, v, mm[r])
                aa[r] = jnp.where(gt, gv, aa[r])
            return tuple(mm) + tuple(aa)

        carry = lax.fori_loop(0, ngroups, body, tuple(ms) + tuple(aas))
        return list(carry[:H]), list(carry[H:])

    ms = [jnp.full((L,), -jnp.inf, dtype=jnp.float32) for _ in range(H)]
    aas = [jnp.zeros((L,), dtype=jnp.int32) for _ in range(H)]

    def round_body(t, carry):
        ms = list(carry[:H])
        aas = list(carry[H:])
        for k in range(RING):
            ci = t * RING + k
            chunk_copy(ci, k).wait()
            ms, aas = scan_groups(buf.at[k], ci * (WC // L), WC // L, ms, aas)

            @pl.when(ci + RING < NCH)
            def _(ci=ci, k=k):
                chunk_copy(ci + RING, k).start()

        return tuple(ms) + tuple(aas)

    carry = lax.fori_loop(0, NROUND, round_body, tuple(ms) + tuple(aas))
    ms, aas = list(carry[:H]), list(carry[H:])

    pltpu.make_async_copy(
        x_hbm.at[b, :, pl.ds(NCH * WC, XTC)], tbuf, semt).wait()
    ms, aas = scan_groups(tbuf, GX, XTC // L, ms, aas)

    for r in range(H):
        mbuf[r] = ms[r]
        abuf[r] = aas[r]

    def head_body(r, resv):
        rm = mbuf[r]
        ra = (abuf[r] << 4) + iota
        for sh in (8, 4, 2, 1):
            perm = iota ^ sh
            mo = rm.at[perm].get(mode="promise_in_bounds")
            ao = ra.at[perm].get(mode="promise_in_bounds")
            better = (mo > rm) | ((mo == rm) & (ao < ra))
            rm = jnp.where(better, mo, rm)
            ra = jnp.where(better, ao, ra)
        val = ra.astype(jnp.float32) / jnp.float32(N)
        return jnp.where(iota == r, val, resv)

    res[...] = lax.fori_loop(0, H, head_body,
                             jnp.zeros((L,), dtype=jnp.float32))
    off = pl.multiple_of(b * L, 8)
    pltpu.sync_copy(res, out_hbm.at[pl.ds(off, L)])


def kernel(x):
    out = _argmax_sc(x)
    return out.reshape(B, L)[:, :H]


# speedup vs baseline: 1.0185x; 1.0185x over previous
import functools

import jax
import jax.numpy as jnp
from jax import lax
from jax.experimental import pallas as pl
from jax.experimental.pallas import tpu as pltpu
from jax.experimental.pallas import tpu_sc as plsc

B, H, N = 32, 8, 100000
L = 16
NC, NS = 2, 16
TB = 128
CTW = 13
WC = CTW * TB
RING = 4
NCH = 60
NROUND = NCH // RING
XTC = N - NCH * WC
GX = (NCH * WC) // L


@functools.partial(
    pl.kernel,
    mesh=plsc.VectorSubcoreMesh(core_axis_name="c", subcore_axis_name="s"),
    out_type=jax.ShapeDtypeStruct((B * L,), jnp.float32),
    scratch_types=[
        pltpu.VMEM((RING, H, WC), jnp.float32),
        pltpu.VMEM((H, XTC), jnp.float32),
        pltpu.VMEM((H, L), jnp.float32),
        pltpu.VMEM((H, L), jnp.int32),
        pltpu.VMEM((L,), jnp.float32),
        pltpu.SemaphoreType.DMA,
        pltpu.SemaphoreType.DMA,
        pltpu.SemaphoreType.DMA,
        pltpu.SemaphoreType.DMA,
        pltpu.SemaphoreType.DMA,
    ],
)
def _argmax_sc(x_hbm, out_hbm, buf, tbuf, mbuf, abuf, res,
               sem0, sem1, sem2, sem3, semt):
    c = lax.axis_index("c")
    s = lax.axis_index("s")
    b = s * NC + c
    sems = (sem0, sem1, sem2, sem3)
    iota = lax.iota(jnp.int32, L)

    def chunk_copy(ci, slot):
        return pltpu.make_async_copy(
            x_hbm.at[b, :, pl.ds(ci * WC, WC)], buf.at[slot], sems[slot])

    for k in range(RING):
        chunk_copy(k, k).start()
    pltpu.make_async_copy(
        x_hbm.at[b, :, pl.ds(NCH * WC, XTC)], tbuf, semt).start()

    def scan_groups(bufref, gbase, ngroups, ms, aas):
        def body(g, carry):
            mm = list(carry[:H])
            aa = list(carry[H:])
            col = g * L
            gv = jnp.broadcast_to(gbase + g, (L,))
            for r in range(H):
                v = bufref[r, pl.ds(col, L)]
                gt = v > mm[r]
                mm[r] = jnp.where(gt, v, mm[r])
                aa[r] = jnp.where(gt, gv, aa[r])
            return tuple(mm) + tuple(aa)

        carry = lax.fori_loop(0, ngroups, body, tuple(ms) + tuple(aas))
        return list(carry[:H]), list(carry[H:])

    ms = [jnp.full((L,), -jnp.inf, dtype=jnp.float32) for _ in range(H)]
    aas = [jnp.zeros((L,), dtype=jnp.int32) for _ in range(H)]

    def round_body(t, carry):
        ms = list(carry[:H])
        aas = list(carry[H:])
        for k in range(RING):
            ci = t * RING + k
            chunk_copy(ci, k).wait()
            ms, aas = scan_groups(buf.at[k], ci * (WC // L), WC // L, ms, aas)

            @pl.when(ci + RING < NCH)
            def _(ci=ci, k=k):
                chunk_copy(ci + RING, k).start()

        return tuple(ms) + tuple(aas)

    carry = lax.fori_loop(0, NROUND, round_body, tuple(ms) + tuple(aas))
    ms, aas = list(carry[:H]), list(carry[H:])

    pltpu.make_async_copy(
        x_hbm.at[b, :, pl.ds(NCH * WC, XTC)], tbuf, semt).wait()
    ms, aas = scan_groups(tbuf, GX, XTC // L, ms, aas)

    for r in range(H):
        mbuf[r] = ms[r]
        abuf[r] = aas[r]

    def head_body(r, resv):
        rm = mbuf[r]
        ra = (abuf[r] << 4) + iota
        for sh in (8, 4, 2, 1):
            perm = iota ^ sh
            mo = rm.at[perm].get(mode="promise_in_bounds")
            ao = ra.at[perm].get(mode="promise_in_bounds")
            better = (mo > rm) | ((mo == rm) & (ao < ra))
            rm = jnp.where(better, mo, rm)
            ra = jnp.where(better, ao, ra)
        val = ra.astype(jnp.float32) / jnp.float32(N)
        return jnp.where(iota == r, val, resv)

    res[...] = lax.fori_loop(0, H, head_body,
                             jnp.zeros((L,), dtype=jnp.float32))
    off = pl.multiple_of(b * L, 8)
    pltpu.sync_copy(res, out_hbm.at[pl.ds(off, L)])


def kernel(x):
    out = _argmax_sc(x)
    return out.reshape(B, L)[:, :H]


# speedup vs baseline: 1.0290x; 1.0103x over previous
import functools

import jax
import jax.numpy as jnp
from jax import lax
from jax.experimental import pallas as pl
from jax.experimental.pallas import tpu as pltpu
from jax.experimental.pallas import tpu_sc as plsc

B, H, N = 32, 8, 100000
L = 16
NC, NS = 2, 16
TB = 128
CTW = 10
WC = CTW * TB
RING = 6
NCH = 78
NROUND = NCH // RING
XTC = N - NCH * WC
GX = (NCH * WC) // L


@functools.partial(
    pl.kernel,
    mesh=plsc.VectorSubcoreMesh(core_axis_name="c", subcore_axis_name="s"),
    out_type=jax.ShapeDtypeStruct((B * L,), jnp.float32),
    scratch_types=[
        pltpu.VMEM((RING, H, WC), jnp.float32),
        pltpu.VMEM((H, XTC), jnp.float32),
        pltpu.VMEM((H, L), jnp.float32),
        pltpu.VMEM((H, L), jnp.int32),
        pltpu.VMEM((L,), jnp.float32),
        pltpu.SemaphoreType.DMA,
        pltpu.SemaphoreType.DMA,
        pltpu.SemaphoreType.DMA,
        pltpu.SemaphoreType.DMA,
        pltpu.SemaphoreType.DMA,
        pltpu.SemaphoreType.DMA,
        pltpu.SemaphoreType.DMA,
    ],
)
def _argmax_sc(x_hbm, out_hbm, buf, tbuf, mbuf, abuf, res,
               sem0, sem1, sem2, sem3, sem4, sem5, semt):
    c = lax.axis_index("c")
    s = lax.axis_index("s")
    b = s * NC + c
    sems = (sem0, sem1, sem2, sem3, sem4, sem5)
    iota = lax.iota(jnp.int32, L)

    def chunk_copy(ci, slot):
        return pltpu.make_async_copy(
            x_hbm.at[b, :, pl.ds(ci * WC, WC)], buf.at[slot], sems[slot])

    for k in range(RING):
        chunk_copy(k, k).start()
    pltpu.make_async_copy(
        x_hbm.at[b, :, pl.ds(NCH * WC, XTC)], tbuf, semt).start()

    def scan_groups(bufref, gbase, ngroups, ms, aas):
        def body(g, carry):
            mm = list(carry[:H])
            aa = list(carry[H:])
            col = g * L
            gv = jnp.broadcast_to(gbase + g, (L,))
            for r in range(H):
                v = bufref[r, pl.ds(col, L)]
                gt = v > mm[r]
                mm[r] = jnp.where(gt, v, mm[r])
                aa[r] = jnp.where(gt, gv, aa[r])
            return tuple(mm) + tuple(aa)

        carry = lax.fori_loop(0, ngroups, body, tuple(ms) + tuple(aas))
        return list(carry[:H]), list(carry[H:])

    ms = [jnp.full((L,), -jnp.inf, dtype=jnp.float32) for _ in range(H)]
    aas = [jnp.zeros((L,), dtype=jnp.int32) for _ in range(H)]

    def round_body(t, carry):
        ms = list(carry[:H])
        aas = list(carry[H:])
        for k in range(RING):
            ci = t * RING + k
            chunk_copy(ci, k).wait()
            ms, aas = scan_groups(buf.at[k], ci * (WC // L), WC // L, ms, aas)

            @pl.when(ci + RING < NCH)
            def _(ci=ci, k=k):
                chunk_copy(ci + RING, k).start()

        return tuple(ms) + tuple(aas)

    carry = lax.fori_loop(0, NROUND, round_body, tuple(ms) + tuple(aas))
    ms, aas = list(carry[:H]), list(carry[H:])

    pltpu.make_async_copy(
        x_hbm.at[b, :, pl.ds(NCH * WC, XTC)], tbuf, semt).wait()
    ms, aas = scan_groups(tbuf, GX, XTC // L, ms, aas)

    for r in range(H):
        mbuf[r] = ms[r]
        abuf[r] = aas[r]

    def head_body(r, resv):
        rm = mbuf[r]
        ra = (abuf[r] << 4) + iota
        for sh in (8, 4, 2, 1):
            perm = iota ^ sh
            mo = rm.at[perm].get(mode="promise_in_bounds")
            ao = ra.at[perm].get(mode="promise_in_bounds")
            better = (mo > rm) | ((mo == rm) & (ao < ra))
            rm = jnp.where(better, mo, rm)
            ra = jnp.where(better, ao, ra)
        val = ra.astype(jnp.float32) / jnp.float32(N)
        return jnp.where(iota == r, val, resv)

    res[...] = lax.fori_loop(0, H, head_body,
                             jnp.zeros((L,), dtype=jnp.float32))
    off = pl.multiple_of(b * L, 8)
    pltpu.sync_copy(res, out_hbm.at[pl.ds(off, L)])


def kernel(x):
    out = _argmax_sc(x)
    return out.reshape(B, L)[:, :H]
